# baseline (speedup 1.0000x reference)
import functools

import jax
import jax.numpy as jnp
from jax import lax
from jax.experimental import pallas as pl
from jax.experimental.pallas import tpu as pltpu
from jax.experimental.pallas import tpu_sc as plsc

BATCH = 16384
D = 128
L = 16
NC = 2
NS = 16
NW = NC * NS
ROWS_PER_W = BATCH // NW
CHUNK = 128
NCHUNK = ROWS_PER_W // CHUNK


def _neg_sqrt(s):
    sc = jnp.maximum(s, jnp.float32(1e-30))
    ix = lax.bitcast_convert_type(sc, jnp.int32)
    iy = jnp.int32(0x5F3759DF) - lax.shift_right_arithmetic(ix, 1)
    y = lax.bitcast_convert_type(iy, jnp.float32)
    half = jnp.float32(0.5) * sc
    for _ in range(3):
        y = y * (jnp.float32(1.5) - half * y * y)
    return -(sc * y)


def _sc_body(hidx_hbm, ridx_hbm, tidx_hbm, node_hbm, re_hbm, out_hbm,
             hidx_v, ridx_v, tidx_v, hbuf, rbuf, tbuf, accs_v, shared_v, accsT_v, out_v, sem):
    wid = lax.axis_index("s") * NC + lax.axis_index("c")

    for c in range(NCHUNK):
        base = wid * ROWS_PER_W + c * CHUNK
        pltpu.sync_copy(hidx_hbm.at[pl.ds(base, CHUNK)], hidx_v)
        pltpu.sync_copy(ridx_hbm.at[pl.ds(base, CHUNK)], ridx_v)
        pltpu.sync_copy(tidx_hbm.at[pl.ds(base, CHUNK)], tidx_v)

        ch = pltpu.async_copy(node_hbm.at[hidx_v], hbuf, sem)
        cr = pltpu.async_copy(re_hbm.at[ridx_v], rbuf, sem)
        ct = pltpu.async_copy(node_hbm.at[tidx_v], tbuf, sem)
        ch.wait()
        cr.wait()
        ct.wait()

        def row(i, _):
            acc = jnp.zeros((L,), jnp.float32)
            for j in range(D // L):
                sl = pl.ds(j * L, L)
                d = hbuf[i, sl] + rbuf[i, sl] - tbuf[i, sl]
                acc = acc + d * d
            accs_v[i, :] = acc
            return 0

        lax.fori_loop(0, CHUNK, row, 0)

        sid = lax.axis_index("s")
        for k in range(L):
            pltpu.sync_copy(accs_v.at[:, k], shared_v.at[sid, k, :])
        pltpu.sync_copy(shared_v.at[sid], accsT_v)
        for g in range(CHUNK // L):
            sl = pl.ds(g * L, L)
            total = accsT_v[0, sl]
            for k in range(1, L):
                total = total + accsT_v[k, sl]
            out_v[sl] = _neg_sqrt(total)

        pltpu.sync_copy(out_v, out_hbm.at[pl.ds(base, CHUNK)])


@jax.jit
def _run(hidx, ridx, tidx, node_embedding, node_re_embedding):
    mesh = plsc.VectorSubcoreMesh(core_axis_name="c", subcore_axis_name="s")
    scores = pl.kernel(
        _sc_body,
        out_type=jax.ShapeDtypeStruct((BATCH,), jnp.float32),
        mesh=mesh,
        scratch_types=[
            pltpu.VMEM((CHUNK,), jnp.int32),
            pltpu.VMEM((CHUNK,), jnp.int32),
            pltpu.VMEM((CHUNK,), jnp.int32),
            pltpu.VMEM((CHUNK, D), jnp.float32),
            pltpu.VMEM((CHUNK, D), jnp.float32),
            pltpu.VMEM((CHUNK, D), jnp.float32),
            pltpu.VMEM((CHUNK, L), jnp.float32),
            pltpu.VMEM_SHARED((NS, L, CHUNK), jnp.float32),
            pltpu.VMEM((L, CHUNK), jnp.float32),
            pltpu.VMEM((CHUNK,), jnp.float32),
            pltpu.SemaphoreType.DMA,
        ],
    )(hidx, ridx, tidx, node_embedding, node_re_embedding)
    return scores.reshape(BATCH, 1)


def kernel(sample, node_embedding, node_re_embedding):
    hidx = sample[:, 0].astype(jnp.int32)
    ridx = sample[:, 1].astype(jnp.int32)
    tidx = sample[:, 2].astype(jnp.int32)
    return _run(hidx, ridx, tidx, node_embedding, node_re_embedding)


# speedup vs baseline: 1.2133x; 1.2133x over previous
import jax
import jax.numpy as jnp
from jax import lax
from jax.experimental import pallas as pl
from jax.experimental.pallas import tpu as pltpu
from jax.experimental.pallas import tpu_sc as plsc

BATCH = 16384
D = 128
L = 16
NC = 2
NS = 16
NW = NC * NS
ROWS_PER_W = BATCH // NW
CHUNK = 128
NCHUNK = ROWS_PER_W // CHUNK


def _neg_sqrt(s):
    sc = jnp.maximum(s, jnp.float32(1e-30))
    ix = lax.bitcast_convert_type(sc, jnp.int32)
    iy = jnp.int32(0x5F3759DF) - lax.shift_right_arithmetic(ix, 1)
    y = lax.bitcast_convert_type(iy, jnp.float32)
    half = jnp.float32(0.5) * sc
    for _ in range(3):
        y = y * (jnp.float32(1.5) - half * y * y)
    return -(sc * y)


def _sc_body(hidx_hbm, ridx_hbm, tidx_hbm, node_hbm, re_hbm, out_hbm,
             hidx_v, ridx_v, tidx_v,
             hb0, rb0, tb0, hb1, rb1, tb1,
             accs_v, shared_v, accsT_v, out_v,
             isem, gsem0, gsem1):
    wid = lax.axis_index("s") * NC + lax.axis_index("c")
    sid = lax.axis_index("s")
    wbase = wid * ROWS_PER_W

    wsl = pl.ds(wbase, ROWS_PER_W)
    di = pltpu.async_copy(hidx_hbm.at[wsl], hidx_v, isem)
    dr = pltpu.async_copy(ridx_hbm.at[wsl], ridx_v, isem)
    dt = pltpu.async_copy(tidx_hbm.at[wsl], tidx_v, isem)
    di.wait()
    dr.wait()
    dt.wait()

    bufs = ((hb0, rb0, tb0, gsem0), (hb1, rb1, tb1, gsem1))

    def start(c):
        hb, rb, tb, sem = bufs[c % 2]
        csl = pl.ds(c * CHUNK, CHUNK)
        return (pltpu.async_copy(node_hbm.at[hidx_v.at[csl]], hb, sem),
                pltpu.async_copy(re_hbm.at[ridx_v.at[csl]], rb, sem),
                pltpu.async_copy(node_hbm.at[tidx_v.at[csl]], tb, sem))

    pending = start(0)
    for c in range(NCHUNK):
        hb, rb, tb, _ = bufs[c % 2]
        for d in pending:
            d.wait()
        if c + 1 < NCHUNK:
            pending = start(c + 1)

        def row(i, _):
            acc = jnp.zeros((L,), jnp.float32)
            for j in range(D // L):
                sl = pl.ds(j * L, L)
                d = hb[i, sl] + rb[i, sl] - tb[i, sl]
                acc = acc + d * d
            accs_v[i, :] = acc
            return 0

        lax.fori_loop(0, CHUNK, row, 0)

        for k in range(L):
            pltpu.sync_copy(accs_v.at[:, k], shared_v.at[sid, k, :])
        pltpu.sync_copy(shared_v.at[sid], accsT_v)
        for g in range(CHUNK // L):
            sl = pl.ds(g * L, L)
            total = accsT_v[0, sl]
            for k in range(1, L):
                total = total + accsT_v[k, sl]
            out_v[sl] = _neg_sqrt(total)

        pltpu.sync_copy(out_v, out_hbm.at[pl.ds(wbase + c * CHUNK, CHUNK)])


@jax.jit
def _run(hidx, ridx, tidx, node_embedding, node_re_embedding):
    mesh = plsc.VectorSubcoreMesh(core_axis_name="c", subcore_axis_name="s")
    return pl.kernel(
        _sc_body,
        out_type=jax.ShapeDtypeStruct((BATCH,), jnp.float32),
        mesh=mesh,
        scratch_types=[
            pltpu.VMEM((ROWS_PER_W,), jnp.int32),
            pltpu.VMEM((ROWS_PER_W,), jnp.int32),
            pltpu.VMEM((ROWS_PER_W,), jnp.int32),
            pltpu.VMEM((CHUNK, D), jnp.float32),
            pltpu.VMEM((CHUNK, D), jnp.float32),
            pltpu.VMEM((CHUNK, D), jnp.float32),
            pltpu.VMEM((CHUNK, D), jnp.float32),
            pltpu.VMEM((CHUNK, D), jnp.float32),
            pltpu.VMEM((CHUNK, D), jnp.float32),
            pltpu.VMEM((CHUNK, L), jnp.float32),
            pltpu.VMEM_SHARED((NS, L, CHUNK), jnp.float32),
            pltpu.VMEM((L, CHUNK), jnp.float32),
            pltpu.VMEM((CHUNK,), jnp.float32),
            pltpu.SemaphoreType.DMA,
            pltpu.SemaphoreType.DMA,
            pltpu.SemaphoreType.DMA,
        ],
    )(hidx, ridx, tidx, node_embedding, node_re_embedding).reshape(BATCH, 1)


def kernel(sample, node_embedding, node_re_embedding):
    sample = sample.astype(jnp.int32)
    return _run(sample[:, 0], sample[:, 1], sample[:, 2],
                node_embedding, node_re_embedding)


# speedup vs baseline: 1.2240x; 1.0088x over previous
import jax
import jax.numpy as jnp
from jax import lax
from jax.experimental import pallas as pl
from jax.experimental.pallas import tpu as pltpu
from jax.experimental.pallas import tpu_sc as plsc

BATCH = 16384
D = 128
L = 16
NC = 2
NS = 16
NW = NC * NS
ROWS_PER_W = BATCH // NW
CHUNK = 64
NCHUNK = ROWS_PER_W // CHUNK


def _neg_sqrt(s):
    sc = jnp.maximum(s, jnp.float32(1e-30))
    ix = lax.bitcast_convert_type(sc, jnp.int32)
    iy = jnp.int32(0x5F3759DF) - lax.shift_right_arithmetic(ix, 1)
    y = lax.bitcast_convert_type(iy, jnp.float32)
    half = jnp.float32(0.5) * sc
    for _ in range(3):
        y = y * (jnp.float32(1.5) - half * y * y)
    return -(sc * y)


def _sc_body(hidx_hbm, ridx_hbm, tidx_hbm, node_hbm, re_hbm, out_hbm,
             hidx_v, ridx_v, tidx_v,
             hb0, rb0, tb0, hb1, rb1, tb1,
             accs0_v, accs1_v, shared_v, accsT_v, out_v,
             isem, gsem0, gsem1, tsem):
    wid = lax.axis_index("s") * NC + lax.axis_index("c")
    sid = lax.axis_index("s")
    wbase = wid * ROWS_PER_W

    wsl = pl.ds(wbase, ROWS_PER_W)
    di = pltpu.async_copy(hidx_hbm.at[wsl], hidx_v, isem)
    dr = pltpu.async_copy(ridx_hbm.at[wsl], ridx_v, isem)
    dt = pltpu.async_copy(tidx_hbm.at[wsl], tidx_v, isem)
    di.wait()
    dr.wait()
    dt.wait()

    bufs = ((hb0, rb0, tb0, gsem0), (hb1, rb1, tb1, gsem1))

    def start(c):
        hb, rb, tb, sem = bufs[c % 2]
        csl = pl.ds(c * CHUNK, CHUNK)
        return (pltpu.async_copy(node_hbm.at[hidx_v.at[csl]], hb, sem),
                pltpu.async_copy(re_hbm.at[ridx_v.at[csl]], rb, sem),
                pltpu.async_copy(node_hbm.at[tidx_v.at[csl]], tb, sem))

    accs = (accs0_v, accs1_v)

    def drain(c):
        for d in cols[c]:
            d.wait()
        pltpu.sync_copy(shared_v.at[pl.ds(sid * L, L), :], accsT_v)
        for g in range(CHUNK // L):
            sl = pl.ds(g * L, L)
            total = accsT_v[0, sl]
            for k in range(1, L):
                total = total + accsT_v[k, sl]
            out_v[pl.ds(c * CHUNK + g * L, L)] = _neg_sqrt(total)

    cols = {}
    pending = start(0)
    for c in range(NCHUNK):
        hb, rb, tb, _ = bufs[c % 2]
        for d in pending:
            d.wait()
        if c + 1 < NCHUNK:
            pending = start(c + 1)
        if c > 0:
            drain(c - 1)

        av = accs[c % 2]

        def row(i, _):
            acc = jnp.zeros((L,), jnp.float32)
            for j in range(D // L):
                sl = pl.ds(j * L, L)
                d = hb[i, sl] + rb[i, sl] - tb[i, sl]
                acc = acc + d * d
            av[i, :] = acc
            return 0

        lax.fori_loop(0, CHUNK, row, 0)

        cols[c] = [pltpu.async_copy(av.at[:, k], shared_v.at[sid * L + k, :], tsem)
                   for k in range(L)]

    drain(NCHUNK - 1)
    pltpu.sync_copy(out_v, out_hbm.at[wsl])


@jax.jit
def _run(hidx, ridx, tidx, node_embedding, node_re_embedding):
    mesh = plsc.VectorSubcoreMesh(core_axis_name="c", subcore_axis_name="s")
    return pl.kernel(
        _sc_body,
        out_type=jax.ShapeDtypeStruct((BATCH,), jnp.float32),
        mesh=mesh,
        scratch_types=[
            pltpu.VMEM((ROWS_PER_W,), jnp.int32),
            pltpu.VMEM((ROWS_PER_W,), jnp.int32),
            pltpu.VMEM((ROWS_PER_W,), jnp.int32),
            pltpu.VMEM((CHUNK, D), jnp.float32),
            pltpu.VMEM((CHUNK, D), jnp.float32),
            pltpu.VMEM((CHUNK, D), jnp.float32),
            pltpu.VMEM((CHUNK, D), jnp.float32),
            pltpu.VMEM((CHUNK, D), jnp.float32),
            pltpu.VMEM((CHUNK, D), jnp.float32),
            pltpu.VMEM((CHUNK, L), jnp.float32),
            pltpu.VMEM((CHUNK, L), jnp.float32),
            pltpu.VMEM_SHARED((NS * L, CHUNK), jnp.float32),
            pltpu.VMEM((L, CHUNK), jnp.float32),
            pltpu.VMEM((ROWS_PER_W,), jnp.float32),
            pltpu.SemaphoreType.DMA,
            pltpu.SemaphoreType.DMA,
            pltpu.SemaphoreType.DMA,
            pltpu.SemaphoreType.DMA,
        ],
    )(hidx, ridx, tidx, node_embedding, node_re_embedding).reshape(BATCH, 1)


def kernel(sample, node_embedding, node_re_embedding):
    sample = sample.astype(jnp.int32)
    return _run(sample[:, 0], sample[:, 1], sample[:, 2],
                node_embedding, node_re_embedding)


# speedup vs baseline: 1.2743x; 1.0410x over previous
import jax
import jax.numpy as jnp
from jax import lax
from jax.experimental import pallas as pl
from jax.experimental.pallas import tpu as pltpu
from jax.experimental.pallas import tpu_sc as plsc

BATCH = 16384
D = 128
L = 16
NC = 2
NS = 16
NW = NC * NS
ROWS_PER_W = BATCH // NW
CHUNK = 128
NCHUNK = ROWS_PER_W // CHUNK


def _neg_sqrt(s):
    sc = jnp.maximum(s, jnp.float32(1e-30))
    ix = lax.bitcast_convert_type(sc, jnp.int32)
    iy = jnp.int32(0x5F3759DF) - lax.shift_right_arithmetic(ix, 1)
    y = lax.bitcast_convert_type(iy, jnp.float32)
    half = jnp.float32(0.5) * sc
    for _ in range(3):
        y = y * (jnp.float32(1.5) - half * y * y)
    return -(sc * y)


def _sc_body(hidx_hbm, ridx_hbm, tidx_hbm, node_hbm, re_hbm, out_hbm,
             hidx_v, ridx_v, tidx_v,
             hb0, rb0, tb0, hb1, rb1, tb1,
             accs0_v, shared_v, accsT_v, out_v,
             isem, gsem0, gsem1, tsem):
    wid = lax.axis_index("s") * NC + lax.axis_index("c")
    sid = lax.axis_index("s")
    wbase = wid * ROWS_PER_W

    wsl = pl.ds(wbase, ROWS_PER_W)
    di = pltpu.async_copy(hidx_hbm.at[wsl], hidx_v, isem)
    dr = pltpu.async_copy(ridx_hbm.at[wsl], ridx_v, isem)
    dt = pltpu.async_copy(tidx_hbm.at[wsl], tidx_v, isem)
    di.wait()
    dr.wait()
    dt.wait()

    bufs = ((hb0, rb0, tb0, gsem0), (hb1, rb1, tb1, gsem1))

    def start(c):
        hb, rb, tb, sem = bufs[c % 2]
        csl = pl.ds(c * CHUNK, CHUNK)
        return (pltpu.async_copy(node_hbm.at[hidx_v.at[csl]], hb, sem),
                pltpu.async_copy(re_hbm.at[ridx_v.at[csl]], rb, sem),
                pltpu.async_copy(node_hbm.at[tidx_v.at[csl]], tb, sem))

    accs = (accs0_v, accs0_v)

    def drain(c):
        for d in cols[c]:
            d.wait()
        pltpu.sync_copy(shared_v.at[sid], accsT_v)
        for g in range(CHUNK // L):
            sl = pl.ds(g * L, L)
            total = accsT_v[0, sl]
            for k in range(1, L):
                total = total + accsT_v[k, sl]
            out_v[pl.ds(c * CHUNK + g * L, L)] = _neg_sqrt(total)

    cols = {}
    pending = start(0)
    for c in range(NCHUNK):
        hb, rb, tb, _ = bufs[c % 2]
        for d in pending:
            d.wait()
        if c + 1 < NCHUNK:
            pending = start(c + 1)

        av = accs[c % 2]

        def row(i, _):
            acc = jnp.zeros((L,), jnp.float32)
            for j in range(D // L):
                sl = pl.ds(j * L, L)
                d = hb[i, sl] + rb[i, sl] - tb[i, sl]
                acc = acc + d * d
            av[i, :] = acc
            return 0

        lax.fori_loop(0, CHUNK, row, 0)

        cols[c] = [pltpu.async_copy(av.at[:, k], shared_v.at[sid, k, :], tsem)
                   for k in range(L)]
        drain(c)

    pltpu.sync_copy(out_v, out_hbm.at[wsl])


@jax.jit
def _run(hidx, ridx, tidx, node_embedding, node_re_embedding):
    mesh = plsc.VectorSubcoreMesh(core_axis_name="c", subcore_axis_name="s")
    return pl.kernel(
        _sc_body,
        out_type=jax.ShapeDtypeStruct((BATCH,), jnp.float32),
        mesh=mesh,
        scratch_types=[
            pltpu.VMEM((ROWS_PER_W,), jnp.int32),
            pltpu.VMEM((ROWS_PER_W,), jnp.int32),
            pltpu.VMEM((ROWS_PER_W,), jnp.int32),
            pltpu.VMEM((CHUNK, D), jnp.float32),
            pltpu.VMEM((CHUNK, D), jnp.float32),
            pltpu.VMEM((CHUNK, D), jnp.float32),
            pltpu.VMEM((CHUNK, D), jnp.float32),
            pltpu.VMEM((CHUNK, D), jnp.float32),
            pltpu.VMEM((CHUNK, D), jnp.float32),
            pltpu.VMEM((CHUNK, L), jnp.float32),
            pltpu.VMEM_SHARED((NS, L, CHUNK), jnp.float32),
            pltpu.VMEM((L, CHUNK), jnp.float32),
            pltpu.VMEM((ROWS_PER_W,), jnp.float32),
            pltpu.SemaphoreType.DMA,
            pltpu.SemaphoreType.DMA,
            pltpu.SemaphoreType.DMA,
            pltpu.SemaphoreType.DMA,
        ],
    )(hidx, ridx, tidx, node_embedding, node_re_embedding).reshape(BATCH, 1)


def kernel(sample, node_embedding, node_re_embedding):
    sample = sample.astype(jnp.int32)
    return _run(sample[:, 0], sample[:, 1], sample[:, 2],
                node_embedding, node_re_embedding)
